# baseline (speedup 1.0000x reference)
import functools

import jax
import jax.numpy as jnp
from jax import lax
from jax.experimental import pallas as pl
from jax.experimental.pallas import tpu as pltpu
from jax.experimental.pallas import tpu_sc as plsc

B = 4
H = 384
W = 384
C = 96
CP = 128
NW = 32
WPB = NW // B
ROWS_PER_W = H * B // NW
CHUNK_PX = 32
CHUNKS_PER_ROW = W // CHUNK_PX
NCHUNK = ROWS_PER_W * CHUNKS_PER_ROW
CVEC = C // 16
NSLOT = 3


def _bcast_lane(v, lane):
    idx = jnp.full((16,), lane, jnp.int32)
    return v.at[idx].get(mode="promise_in_bounds")


def _body(im_hbm, aff_hbm, ys_hbm, xs_hbm, out_hbm,
          aff_v, ys_v, xs_v,
          idx0_v, idx1_v, idx2_v, rows0_v, rows1_v, rows2_v,
          w0_v, w1_v, w2_v, outb0_v, outb1_v, outb2_v,
          sem_g0, sem_g1, sem_g2, sem_o0, sem_o1, sem_o2):
    idx_s = (idx0_v, idx1_v, idx2_v)
    rows_s = (rows0_v, rows1_v, rows2_v)
    w_s = (w0_v, w1_v, w2_v)
    outb_s = (outb0_v, outb1_v, outb2_v)
    sem_g = (sem_g0, sem_g1, sem_g2)
    sem_o = (sem_o0, sem_o1, sem_o2)

    cid = lax.axis_index("c")
    sid = lax.axis_index("s")
    wid = sid * 2 + cid
    b = wid // WPB
    i0 = (wid % WPB) * ROWS_PER_W
    bbase = b * (H * W)

    pltpu.sync_copy(aff_hbm, aff_v)
    pltpu.sync_copy(xs_hbm, xs_v)
    pltpu.sync_copy(ys_hbm.at[pl.ds(i0, ROWS_PER_W)], ys_v)

    a00 = aff_v[b * 6 + 0, :]
    a01 = aff_v[b * 6 + 1, :]
    a02 = aff_v[b * 6 + 2, :]
    a10 = aff_v[b * 6 + 3, :]
    a11 = aff_v[b * 6 + 4, :]
    a12 = aff_v[b * 6 + 5, :]

    def compute_and_fire(k, s):
        idx_v, rows_v, w_v = idx_s[s], rows_s[s], w_s[s]
        r = k // CHUNKS_PER_ROW
        q = k % CHUNKS_PER_ROW
        yv = ys_v[r, :]
        t0 = a01 * yv + a02
        t1 = a11 * yv + a12
        for g in range(2):
            xs16 = xs_v[q * 2 + g, :]
            xn = a00 * xs16 + t0
            yn = a10 * xs16 + t1
            x = (xn + 1.0) * (0.5 * W)
            y = (yn + 1.0) * (0.5 * H)
            xl = jnp.clip(x.astype(jnp.int32), 0, W - 1)
            xr = jnp.minimum(xl + 1, W - 1)
            yu = jnp.clip(y.astype(jnp.int32), 0, H - 1)
            yb = jnp.minimum(yu + 1, H - 1)
            ilu = bbase + yu * W + xl
            ilb = bbase + yb * W + xl
            dxi = xr - xl
            idx_v[pl.ds(g * 64 + 0, 16)] = ilu
            idx_v[pl.ds(g * 64 + 16, 16)] = ilb
            idx_v[pl.ds(g * 64 + 32, 16)] = ilu + dxi
            idx_v[pl.ds(g * 64 + 48, 16)] = ilb + dxi
            dxl = x - xl.astype(jnp.float32)
            dxr = xr.astype(jnp.float32) - x
            dyu = y - yu.astype(jnp.float32)
            dyb = yb.astype(jnp.float32) - y
            wlu = jnp.exp(-dxl * dyu)
            wlb = jnp.exp(-dxl * dyb)
            wru = jnp.exp(-dxr * dyu)
            wrb = jnp.exp(-dxr * dyb)
            inv = 1.0 / (wlu + wlb + wru + wrb)
            w_v[g * 64 + 0, :] = wlu * inv
            w_v[g * 64 + 16, :] = wlb * inv
            w_v[g * 64 + 32, :] = wru * inv
            w_v[g * 64 + 48, :] = wrb * inv
        pltpu.make_async_copy(im_hbm.at[idx_v], rows_v, sem_g[s]).start()

    def wait_gather(s):
        pltpu.make_async_copy(im_hbm.at[idx_s[s]], rows_s[s], sem_g[s]).wait()

    def combine_and_fire(k, s):
        rows_v, w_v, outb_v = rows_s[s], w_s[s], outb_s[s]
        for g in range(2):
            wlu16 = w_v[g * 64 + 0, :]
            wlb16 = w_v[g * 64 + 16, :]
            wru16 = w_v[g * 64 + 32, :]
            wrb16 = w_v[g * 64 + 48, :]

            def lane4_body(lb, _):
                def flush(st):
                    p_prev, halfs = st
                    for cc in range(CVEC):
                        outb_v[p_prev, pl.ds(cc * 16, 16)] = (
                            halfs[cc][0] + halfs[cc][1])

                prev = None
                for u in range(4):
                    lane = lb * 4 + u
                    r0 = g * 64 + lane
                    p = g * 16 + lane
                    wlu = _bcast_lane(wlu16, lane)
                    wlb = _bcast_lane(wlb16, lane)
                    wru = _bcast_lane(wru16, lane)
                    wrb = _bcast_lane(wrb16, lane)
                    vals = [[rows_v[r0 + 16 * t, pl.ds(cc * 16, 16)]
                             for t in range(4)] for cc in range(CVEC)]
                    halfs = [(wlu * v[0] + wlb * v[1], wru * v[2] + wrb * v[3])
                             for v in vals]
                    if prev is not None:
                        flush(prev)
                    prev = (p, halfs)
                flush(prev)
                return 0
            lax.fori_loop(0, 4, lane4_body, 0)
        r = k // CHUNKS_PER_ROW
        q = k % CHUNKS_PER_ROW
        pltpu.make_async_copy(
            outb_v, out_hbm.at[b, i0 + r, pl.ds(q * CHUNK_PX, CHUNK_PX)],
            sem_o[s]).start()

    def wait_out(s):
        pltpu.make_async_copy(outb_s[s], out_hbm.at[0, 0, pl.ds(0, CHUNK_PX)],
                              sem_o[s]).wait()

    compute_and_fire(jnp.int32(0), 0)
    compute_and_fire(jnp.int32(1), 1)

    def k3_body(k3, _):
        for off in range(NSLOT):
            k = NSLOT * k3 + off
            s = off
            wait_gather(s)
            pl.when(k + 2 < NCHUNK)(
                lambda: compute_and_fire(k + 2, (off + 2) % NSLOT))
            pl.when(k >= NSLOT)(lambda: wait_out(s))
            combine_and_fire(k, s)
        return 0

    lax.fori_loop(0, NCHUNK // NSLOT, k3_body, 0)
    wait_out(0)
    wait_out(1)
    wait_out(2)


@functools.partial(jax.jit, static_argnames=())
def _affine_sample(im_flat, aff_bc, ys_bc, xs_bc):
    mesh = plsc.VectorSubcoreMesh(core_axis_name="c", subcore_axis_name="s",
                                  num_cores=2)
    f = pl.kernel(
        _body,
        out_type=jax.ShapeDtypeStruct((B, H, W, C), jnp.float32),
        mesh=mesh,
        compiler_params=pltpu.CompilerParams(use_tc_tiling_on_sc=True),
        scratch_types=(
            [pltpu.VMEM((B * 6 + 2, 16), jnp.float32),
             pltpu.VMEM((ROWS_PER_W, 16), jnp.float32),
             pltpu.VMEM((W // 16, 16), jnp.float32)]
            + [pltpu.VMEM((4 * CHUNK_PX,), jnp.int32)] * NSLOT
            + [pltpu.VMEM((4 * CHUNK_PX, CP), jnp.float32)] * NSLOT
            + [pltpu.VMEM((4 * CHUNK_PX, 16), jnp.float32)] * NSLOT
            + [pltpu.VMEM((CHUNK_PX, C), jnp.float32)] * NSLOT
            + [pltpu.SemaphoreType.DMA] * (2 * NSLOT)
        ),
    )
    return f(im_flat, aff_bc, ys_bc, xs_bc)


def kernel(inputs, affines):
    zero = (affines[0, 0, 0] * 0.0).astype(jnp.float32)
    im_flat = jnp.pad(inputs.reshape(B * H * W, C) + zero,
                      ((0, 0), (0, CP - C)))
    aff_flat = jnp.concatenate([affines.reshape(-1).astype(jnp.float32),
                                jnp.zeros((2,), jnp.float32)])
    aff_bc = jnp.broadcast_to(aff_flat[:, None], (B * 6 + 2, 16))
    xs = jnp.linspace(-1.0, 1.0, W, dtype=jnp.float32)
    ys = jnp.linspace(-1.0, 1.0, H, dtype=jnp.float32)
    xs_bc = xs.reshape(W // 16, 16)
    ys_bc = jnp.broadcast_to(ys[:, None], (H, 16))
    return _affine_sample(im_flat, aff_bc, ys_bc, xs_bc)


# speedup vs baseline: 1.1160x; 1.1160x over previous
import functools

import jax
import jax.numpy as jnp
from jax import lax
from jax.experimental import pallas as pl
from jax.experimental.pallas import tpu as pltpu
from jax.experimental.pallas import tpu_sc as plsc

B = 4
H = 384
W = 384
C = 96
CP = 128
NW = 32
WPB = NW // B
ROWS_PER_W = H * B // NW
CHUNK_PX = 32
CHUNKS_PER_ROW = W // CHUNK_PX
NCHUNK = ROWS_PER_W * CHUNKS_PER_ROW
CVEC = C // 16
NSLOT = 3


def _bcast_lane(v, lane):
    idx = jnp.full((16,), lane, jnp.int32)
    return v.at[idx].get(mode="promise_in_bounds")


def _body(im_hbm, aff_hbm, ys_hbm, xs_hbm, out_hbm,
          aff_v, ys_v, xs_v,
          idx0_v, idx1_v, idx2_v, rows0_v, rows1_v, rows2_v,
          w0_v, w1_v, w2_v, outb0_v, outb1_v, outb2_v,
          sem_g0, sem_g1, sem_g2, sem_o0, sem_o1, sem_o2):
    idx_s = (idx0_v, idx1_v, idx2_v)
    rows_s = (rows0_v, rows1_v, rows2_v)
    w_s = (w0_v, w1_v, w2_v)
    outb_s = (outb0_v, outb1_v, outb2_v)
    sem_g = (sem_g0, sem_g1, sem_g2)
    sem_o = (sem_o0, sem_o1, sem_o2)

    cid = lax.axis_index("c")
    sid = lax.axis_index("s")
    wid = sid * 2 + cid
    b = wid // WPB
    i0 = (wid % WPB) * ROWS_PER_W
    bbase = b * (H * W)

    pltpu.sync_copy(aff_hbm, aff_v)
    pltpu.sync_copy(xs_hbm, xs_v)
    pltpu.sync_copy(ys_hbm.at[pl.ds(i0, ROWS_PER_W)], ys_v)

    a00 = aff_v[b * 6 + 0, :]
    a01 = aff_v[b * 6 + 1, :]
    a02 = aff_v[b * 6 + 2, :]
    a10 = aff_v[b * 6 + 3, :]
    a11 = aff_v[b * 6 + 4, :]
    a12 = aff_v[b * 6 + 5, :]

    def compute_and_fire(k, s):
        idx_v, rows_v, w_v = idx_s[s], rows_s[s], w_s[s]
        r = k // CHUNKS_PER_ROW
        q = k % CHUNKS_PER_ROW
        yv = ys_v[r, :]
        t0 = a01 * yv + a02
        t1 = a11 * yv + a12
        for g in range(2):
            xs16 = xs_v[q * 2 + g, :]
            xn = a00 * xs16 + t0
            yn = a10 * xs16 + t1
            x = (xn + 1.0) * (0.5 * W)
            y = (yn + 1.0) * (0.5 * H)
            xl = jnp.clip(x.astype(jnp.int32), 0, W - 1)
            xr = jnp.minimum(xl + 1, W - 1)
            yu = jnp.clip(y.astype(jnp.int32), 0, H - 1)
            yb = jnp.minimum(yu + 1, H - 1)
            ilu = bbase + yu * W + xl
            ilb = bbase + yb * W + xl
            dxi = xr - xl
            idx_v[pl.ds(g * 64 + 0, 16)] = ilu
            idx_v[pl.ds(g * 64 + 16, 16)] = ilb
            idx_v[pl.ds(g * 64 + 32, 16)] = ilu + dxi
            idx_v[pl.ds(g * 64 + 48, 16)] = ilb + dxi
            dxl = x - xl.astype(jnp.float32)
            dxr = xr.astype(jnp.float32) - x
            dyu = y - yu.astype(jnp.float32)
            dyb = yb.astype(jnp.float32) - y
            wlu = jnp.exp(-dxl * dyu)
            wlb = jnp.exp(-dxl * dyb)
            wru = jnp.exp(-dxr * dyu)
            wrb = jnp.exp(-dxr * dyb)
            inv = 1.0 / (wlu + wlb + wru + wrb)
            w_v[g * 64 + 0, :] = wlu * inv
            w_v[g * 64 + 16, :] = wlb * inv
            w_v[g * 64 + 32, :] = wru * inv
            w_v[g * 64 + 48, :] = wrb * inv
        pltpu.make_async_copy(im_hbm.at[idx_v], rows_v, sem_g[s]).start()

    def wait_gather(s):
        pltpu.make_async_copy(im_hbm.at[idx_s[s]], rows_s[s], sem_g[s]).wait()

    def combine_and_fire(k, s):
        rows_v, w_v, outb_v = rows_s[s], w_s[s], outb_s[s]
        for g in range(2):
            wlu16 = w_v[g * 64 + 0, :]
            wlb16 = w_v[g * 64 + 16, :]
            wru16 = w_v[g * 64 + 32, :]
            wrb16 = w_v[g * 64 + 48, :]

            def lane4_body(lb, _):
                def flush(st):
                    p_prev, halfs = st
                    for cc in range(CVEC):
                        outb_v[p_prev, pl.ds(cc * 16, 16)] = (
                            halfs[cc][0] + halfs[cc][1])

                prev = None
                for u in range(4):
                    lane = lb * 4 + u
                    r0 = g * 64 + lane
                    p = g * 16 + lane
                    wlu = _bcast_lane(wlu16, lane)
                    wlb = _bcast_lane(wlb16, lane)
                    wru = _bcast_lane(wru16, lane)
                    wrb = _bcast_lane(wrb16, lane)
                    vals = [[rows_v[r0 + 16 * t, pl.ds(cc * 16, 16)]
                             for t in range(4)] for cc in range(CVEC)]
                    halfs = [(wlu * v[0] + wlb * v[1], wru * v[2] + wrb * v[3])
                             for v in vals]
                    if prev is not None:
                        flush(prev)
                    prev = (p, halfs)
                flush(prev)
                return 0
            lax.fori_loop(0, 4, lane4_body, 0)
        r = k // CHUNKS_PER_ROW
        q = k % CHUNKS_PER_ROW
        pltpu.make_async_copy(
            outb_v, out_hbm.at[b, i0 + r, pl.ds(q * CHUNK_PX, CHUNK_PX)],
            sem_o[s]).start()

    def wait_out(s):
        pltpu.make_async_copy(outb_s[s], out_hbm.at[0, 0, pl.ds(0, CHUNK_PX)],
                              sem_o[s]).wait()

    compute_and_fire(jnp.int32(0), 0)
    compute_and_fire(jnp.int32(1), 1)

    def k3_body(k3, _):
        for off in range(NSLOT):
            k = NSLOT * k3 + off
            s = off
            wait_gather(s)
            pl.when(k + 2 < NCHUNK)(
                lambda: compute_and_fire(k + 2, (off + 2) % NSLOT))
            pl.when(k >= NSLOT)(lambda: wait_out(s))
            combine_and_fire(k, s)
        return 0

    lax.fori_loop(0, NCHUNK // NSLOT, k3_body, 0)
    wait_out(0)
    wait_out(1)
    wait_out(2)


@functools.partial(jax.jit, static_argnames=())
def _affine_sample(im_flat, aff_bc, ys_bc, xs_bc):
    mesh = plsc.VectorSubcoreMesh(core_axis_name="c", subcore_axis_name="s",
                                  num_cores=2)
    f = pl.kernel(
        _body,
        out_type=jax.ShapeDtypeStruct((B, H, W, CP), jnp.float32),
        mesh=mesh,
        compiler_params=pltpu.CompilerParams(use_tc_tiling_on_sc=True),
        scratch_types=(
            [pltpu.VMEM((B * 6 + 2, 16), jnp.float32),
             pltpu.VMEM((ROWS_PER_W, 16), jnp.float32),
             pltpu.VMEM((W // 16, 16), jnp.float32)]
            + [pltpu.VMEM((4 * CHUNK_PX,), jnp.int32)] * NSLOT
            + [pltpu.VMEM((4 * CHUNK_PX, CP), jnp.float32)] * NSLOT
            + [pltpu.VMEM((4 * CHUNK_PX, 16), jnp.float32)] * NSLOT
            + [pltpu.VMEM((CHUNK_PX, CP), jnp.float32)] * NSLOT
            + [pltpu.SemaphoreType.DMA] * (2 * NSLOT)
        ),
    )
    return f(im_flat, aff_bc, ys_bc, xs_bc)


def kernel(inputs, affines):
    im_flat = jnp.pad(inputs.reshape(B * H * W, C), ((0, 0), (0, CP - C)))
    aff_flat = jnp.concatenate([affines.reshape(-1).astype(jnp.float32),
                                jnp.zeros((2,), jnp.float32)])
    aff_bc = jnp.broadcast_to(aff_flat[:, None], (B * 6 + 2, 16))
    xs = jnp.linspace(-1.0, 1.0, W, dtype=jnp.float32)
    ys = jnp.linspace(-1.0, 1.0, H, dtype=jnp.float32)
    xs_bc = xs.reshape(W // 16, 16)
    ys_bc = jnp.broadcast_to(ys[:, None], (H, 16))
    return _affine_sample(im_flat, aff_bc, ys_bc, xs_bc)[..., :C]


# speedup vs baseline: 1.2317x; 1.1037x over previous
import functools

import jax
import jax.numpy as jnp
from jax import lax
from jax.experimental import pallas as pl
from jax.experimental.pallas import tpu as pltpu
from jax.experimental.pallas import tpu_sc as plsc

B = 4
H = 384
W = 384
C = 96
CP = 128
NW = 32
WPB = NW // B
ROWS_PER_W = H * B // NW
CHUNK_PX = 32
CHUNKS_PER_ROW = W // CHUNK_PX
NCHUNK = ROWS_PER_W * CHUNKS_PER_ROW
CVEC = C // 16
NSLOT = 3


def _bcast_lane(v, lane):
    idx = jnp.full((16,), lane, jnp.int32)
    return v.at[idx].get(mode="promise_in_bounds")


def _body(im_hbm, aff_hbm, ys_hbm, xs_hbm, out_hbm, table_hbm,
          aff_v, ys_v, xs_v,
          idx0_v, idx1_v, idx2_v, rows0_v, rows1_v, rows2_v,
          w0_v, w1_v, w2_v, outb0_v, outb1_v, outb2_v,
          sem_g0, sem_g1, sem_g2, sem_o0, sem_o1, sem_o2):
    idx_s = (idx0_v, idx1_v, idx2_v)
    rows_s = (rows0_v, rows1_v, rows2_v)
    w_s = (w0_v, w1_v, w2_v)
    outb_s = (outb0_v, outb1_v, outb2_v)
    sem_g = (sem_g0, sem_g1, sem_g2)
    sem_o = (sem_o0, sem_o1, sem_o2)

    cid = lax.axis_index("c")
    sid = lax.axis_index("s")
    b = cid * 2 + sid // 8
    i0 = (sid % 8) * ROWS_PER_W
    bbase = b * (H * W)

    pltpu.sync_copy(aff_hbm, aff_v)
    pltpu.sync_copy(xs_hbm, xs_v)
    pltpu.sync_copy(ys_hbm.at[pl.ds(i0, ROWS_PER_W)], ys_v)

    a00 = aff_v[b * 6 + 0, :]
    a01 = aff_v[b * 6 + 1, :]
    a02 = aff_v[b * 6 + 2, :]
    a10 = aff_v[b * 6 + 3, :]
    a11 = aff_v[b * 6 + 4, :]
    a12 = aff_v[b * 6 + 5, :]

    def compute_and_fire(k, s):
        idx_v, rows_v, w_v = idx_s[s], rows_s[s], w_s[s]
        r = k // CHUNKS_PER_ROW
        q = k % CHUNKS_PER_ROW
        yv = ys_v[r, :]
        t0 = a01 * yv + a02
        t1 = a11 * yv + a12
        for g in range(2):
            xs16 = xs_v[q * 2 + g, :]
            xn = a00 * xs16 + t0
            yn = a10 * xs16 + t1
            x = (xn + 1.0) * (0.5 * W)
            y = (yn + 1.0) * (0.5 * H)
            xl = jnp.clip(x.astype(jnp.int32), 0, W - 1)
            xr = jnp.minimum(xl + 1, W - 1)
            yu = jnp.clip(y.astype(jnp.int32), 0, H - 1)
            yb = jnp.minimum(yu + 1, H - 1)
            ilu = bbase + yu * W + xl
            ilb = bbase + yb * W + xl
            dxi = xr - xl
            idx_v[pl.ds(g * 64 + 0, 16)] = ilu
            idx_v[pl.ds(g * 64 + 16, 16)] = ilb
            idx_v[pl.ds(g * 64 + 32, 16)] = ilu + dxi
            idx_v[pl.ds(g * 64 + 48, 16)] = ilb + dxi
            dxl = x - xl.astype(jnp.float32)
            dxr = xr.astype(jnp.float32) - x
            dyu = y - yu.astype(jnp.float32)
            dyb = yb.astype(jnp.float32) - y
            wlu = jnp.exp(-dxl * dyu)
            wlb = jnp.exp(-dxl * dyb)
            wru = jnp.exp(-dxr * dyu)
            wrb = jnp.exp(-dxr * dyb)
            inv = 1.0 / (wlu + wlb + wru + wrb)
            w_v[g * 64 + 0, :] = wlu * inv
            w_v[g * 64 + 16, :] = wlb * inv
            w_v[g * 64 + 32, :] = wru * inv
            w_v[g * 64 + 48, :] = wrb * inv
        pltpu.make_async_copy(table_hbm.at[idx_v], rows_v, sem_g[s]).start()

    def wait_gather(s):
        pltpu.make_async_copy(table_hbm.at[idx_s[s]], rows_s[s], sem_g[s]).wait()

    def combine_and_fire(k, s):
        rows_v, w_v, outb_v = rows_s[s], w_s[s], outb_s[s]
        for g in range(2):
            wlu16 = w_v[g * 64 + 0, :]
            wlb16 = w_v[g * 64 + 16, :]
            wru16 = w_v[g * 64 + 32, :]
            wrb16 = w_v[g * 64 + 48, :]

            def lane4_body(lb, _):
                def flush(st):
                    p_prev, halfs = st
                    for cc in range(CVEC):
                        outb_v[p_prev, pl.ds(cc * 16, 16)] = (
                            halfs[cc][0] + halfs[cc][1])

                prev = None
                for u in range(4):
                    lane = lb * 4 + u
                    r0 = g * 64 + lane
                    p = g * 16 + lane
                    wlu = _bcast_lane(wlu16, lane)
                    wlb = _bcast_lane(wlb16, lane)
                    wru = _bcast_lane(wru16, lane)
                    wrb = _bcast_lane(wrb16, lane)
                    vals = [[rows_v[r0 + 16 * t, pl.ds(cc * 16, 16)]
                             for t in range(4)] for cc in range(CVEC)]
                    halfs = [(wlu * v[0] + wlb * v[1], wru * v[2] + wrb * v[3])
                             for v in vals]
                    if prev is not None:
                        flush(prev)
                    prev = (p, halfs)
                flush(prev)
                return 0
            lax.fori_loop(0, 4, lane4_body, 0)
        r = k // CHUNKS_PER_ROW
        q = k % CHUNKS_PER_ROW
        pltpu.make_async_copy(
            outb_v, out_hbm.at[b, i0 + r, pl.ds(q * CHUNK_PX, CHUNK_PX)],
            sem_o[s]).start()

    def wait_out(s):
        pltpu.make_async_copy(outb_s[s], out_hbm.at[0, 0, pl.ds(0, CHUNK_PX)],
                              sem_o[s]).wait()

    compute_and_fire(jnp.int32(0), 0)
    compute_and_fire(jnp.int32(1), 1)

    def k3_body(k3, _):
        for off in range(NSLOT):
            k = NSLOT * k3 + off
            s = off
            wait_gather(s)
            pl.when(k + 2 < NCHUNK)(
                lambda: compute_and_fire(k + 2, (off + 2) % NSLOT))
            pl.when(k >= NSLOT)(lambda: wait_out(s))
            combine_and_fire(k, s)
        return 0

    lax.fori_loop(0, NCHUNK // NSLOT, k3_body, 0)
    wait_out(0)
    wait_out(1)
    wait_out(2)


@functools.partial(jax.jit, static_argnames=())
def _affine_sample(im_flat, aff_bc, ys_bc, xs_bc):
    mesh = plsc.VectorSubcoreMesh(core_axis_name="c", subcore_axis_name="s",
                                  num_cores=2)
    f = pl.kernel(
        _body,
        out_type=(jax.ShapeDtypeStruct((B, H, W, CP), jnp.float32),
                  jax.ShapeDtypeStruct((B * H * W, CP), jnp.float32)),
        mesh=mesh,
        compiler_params=pltpu.CompilerParams(use_tc_tiling_on_sc=True),
        scratch_types=(
            [pltpu.VMEM((B * 6 + 2, 16), jnp.float32),
             pltpu.VMEM((ROWS_PER_W, 16), jnp.float32),
             pltpu.VMEM((W // 16, 16), jnp.float32)]
            + [pltpu.VMEM((4 * CHUNK_PX,), jnp.int32)] * NSLOT
            + [pltpu.VMEM((4 * CHUNK_PX, CP), jnp.float32)] * NSLOT
            + [pltpu.VMEM((4 * CHUNK_PX, 16), jnp.float32)] * NSLOT
            + [pltpu.VMEM((CHUNK_PX, CP), jnp.float32)] * NSLOT
            + [pltpu.SemaphoreType.DMA] * (2 * NSLOT)
        ),
    )
    return f(im_flat, aff_bc, ys_bc, xs_bc)


def kernel(inputs, affines):
    aff_flat = jnp.concatenate([affines.reshape(-1).astype(jnp.float32),
                                jnp.zeros((2,), jnp.float32)])
    aff_bc = jnp.broadcast_to(aff_flat[:, None], (B * 6 + 2, 16))
    xs = jnp.linspace(-1.0, 1.0, W, dtype=jnp.float32)
    ys = jnp.linspace(-1.0, 1.0, H, dtype=jnp.float32)
    xs_bc = xs.reshape(W // 16, 16)
    ys_bc = jnp.broadcast_to(ys[:, None], (H, 16))
    out, _ = _affine_sample(inputs, aff_bc, ys_bc, xs_bc)
    return out[..., :C]
